# baseline (speedup 1.0000x reference)
import functools

import jax
import jax.numpy as jnp
from jax.experimental import pallas as pl


def _body(x0_ref, x1_ref, w_ref, b_ref, o_ref, *, E):
    H = E // 2
    x0 = x0_ref[...].astype(jnp.bfloat16)
    x1 = x1_ref[...].astype(jnp.bfloat16)
    for e in range(E):
        xe = (x0 if e < H else x1)[:, e % H, :]
        we = w_ref[e]
        y = jax.lax.dot_general(
            xe, we, (((1,), (1,)), ((), ())),
            preferred_element_type=jnp.float32,
        )
        o_ref[:, e, :] = y + b_ref[e]


def kernel(inputs, W, b):
    N, E, D = inputs.shape
    BN = 128
    w_bf = W.astype(jnp.bfloat16)

    return pl.pallas_call(
        functools.partial(_body, E=E),
        grid=(N // BN,),
        in_specs=[
            pl.BlockSpec((BN, E // 2, D), lambda i: (i, 0, 0)),
            pl.BlockSpec((BN, E // 2, D), lambda i: (i, 1, 0)),
            pl.BlockSpec((E, D, D), lambda i: (0, 0, 0)),
            pl.BlockSpec((E, D), lambda i: (0, 0)),
        ],
        out_specs=pl.BlockSpec((BN, E, D), lambda i: (i, 0, 0)),
        out_shape=jax.ShapeDtypeStruct((N, E, D), jnp.float32),
    )(inputs, inputs, w_bf, b)


# speedup vs baseline: 1.7662x; 1.7662x over previous
import jax
import jax.numpy as jnp
from jax.experimental import pallas as pl


def _body(x_ref, w_ref, b_ref, o_ref):
    x = x_ref[...].astype(jnp.bfloat16)
    y = jax.lax.dot_general(
        x, w_ref[...],
        (((2,), (2,)), ((1,), (0,))),
        preferred_element_type=jnp.float32,
    )
    o_ref[...] = y.swapaxes(0, 1) + b_ref[...]


def kernel(inputs, W, b):
    N, E, D = inputs.shape
    BN = 256
    w_bf = W.astype(jnp.bfloat16)
    return pl.pallas_call(
        _body,
        grid=(N // BN,),
        in_specs=[
            pl.BlockSpec((BN, E, D), lambda i: (i, 0, 0)),
            pl.BlockSpec((E, D, D), lambda i: (0, 0, 0)),
            pl.BlockSpec((E, D), lambda i: (0, 0)),
        ],
        out_specs=pl.BlockSpec((BN, E, D), lambda i: (i, 0, 0)),
        out_shape=jax.ShapeDtypeStruct((N, E, D), jnp.float32),
    )(inputs, w_bf, b)
